# baseline (speedup 1.0000x reference)
import functools

import jax
import jax.numpy as jnp
from jax import lax
from jax.experimental import pallas as pl
from jax.experimental.pallas import tpu as pltpu
from jax.experimental.pallas import tpu_sc as plsc

_BATCH = 1024
_SEQ = 200
_DIM = 64
_NC = 2
_NS = 16
_NW = _NC * _NS
_ROWS_PER_W = _BATCH * _SEQ // _NW
_CHUNK = 128
_NCHUNK = _ROWS_PER_W // _CHUNK


def _emb_kernel(x_hbm, tok_hbm, pos_hbm, out_hbm,
                idx_v, pos_v, buf0, buf1, sem0, sem1):
    c = lax.axis_index("c")
    s = lax.axis_index("s")
    wid = s * _NC + c
    pltpu.sync_copy(x_hbm.at[wid], idx_v)
    pltpu.sync_copy(pos_hbm.at[pl.ds(0, _SEQ)], pos_v.at[pl.ds(0, _SEQ)])
    pltpu.sync_copy(pos_hbm.at[pl.ds(0, _SEQ)], pos_v.at[pl.ds(_SEQ, _SEQ)])

    bufs = (buf0, buf1)
    sems = (sem0, sem1)

    pltpu.async_copy(tok_hbm.at[idx_v.at[0]], bufs[0], sems[0])

    def process(g, s0, buf, sem):
        pltpu.make_async_copy(tok_hbm.at[idx_v.at[g]], buf, sem).wait()

        def row_add(r, carry):
            for v in range(4):
                plsc.addupdate(buf.at[r, pl.ds(v * 16, 16)],
                               pos_v[s0 + r, pl.ds(v * 16, 16)])
            return carry
        lax.fori_loop(0, _CHUNK, row_add, 0, unroll=4)

        pltpu.sync_copy(buf, out_hbm.at[pl.ds(wid * _ROWS_PER_W + g * _CHUNK,
                                              _CHUNK)])

    def chunk_pair(i, s0):
        g = 2 * i

        pltpu.async_copy(tok_hbm.at[idx_v.at[g + 1]], bufs[1], sems[1])
        process(g, s0, bufs[0], sems[0])
        s0 = s0 + _CHUNK
        s0 = jnp.where(s0 >= _SEQ, s0 - _SEQ, s0)

        @pl.when(g + 2 < _NCHUNK)
        def _():
            pltpu.async_copy(tok_hbm.at[idx_v.at[g + 2]], bufs[0], sems[0])
        process(g + 1, s0, bufs[1], sems[1])
        s0 = s0 + _CHUNK
        s0 = jnp.where(s0 >= _SEQ, s0 - _SEQ, s0)
        return s0

    lax.fori_loop(0, _NCHUNK // 2, chunk_pair, jnp.int32(0))


def kernel(x, token_table, pos_table):
    x3 = x.reshape(_NW, _NCHUNK, _CHUNK).astype(jnp.int32)
    mesh = plsc.VectorSubcoreMesh(core_axis_name="c", subcore_axis_name="s",
                                  num_cores=_NC, num_subcores=_NS)
    run = functools.partial(
        pl.kernel,
        out_type=jax.ShapeDtypeStruct((_BATCH * _SEQ, _DIM), jnp.float32),
        mesh=mesh,
        compiler_params=pltpu.CompilerParams(use_tc_tiling_on_sc=False),
        scratch_types=[
            pltpu.VMEM((_NCHUNK, _CHUNK), jnp.int32),
            pltpu.VMEM((2 * _SEQ, _DIM), jnp.float32),
            pltpu.VMEM((_CHUNK, _DIM), jnp.float32),
            pltpu.VMEM((_CHUNK, _DIM), jnp.float32),
            pltpu.SemaphoreType.DMA,
            pltpu.SemaphoreType.DMA,
        ],
    )(_emb_kernel)
    out = run(x3, token_table, pos_table)
    return out.reshape(_BATCH, _SEQ, _DIM)


# speedup vs baseline: 1.0735x; 1.0735x over previous
import functools

import jax
import jax.numpy as jnp
from jax import lax
from jax.experimental import pallas as pl
from jax.experimental.pallas import tpu as pltpu
from jax.experimental.pallas import tpu_sc as plsc

_BATCH = 1024
_SEQ = 200
_DIM = 64
_NC = 2
_NS = 16
_NW = _NC * _NS
_ROWS_PER_W = _BATCH * _SEQ // _NW
_CHUNK = 640
_NCHUNK = _ROWS_PER_W // _CHUNK


def _gather_kernel(x_hbm, tok_hbm, out_hbm, idx_v, buf0, buf1, sem0, sem1):
    c = lax.axis_index("c")
    s = lax.axis_index("s")
    wid = s * _NC + c
    pltpu.sync_copy(x_hbm.at[wid], idx_v)

    bufs = (buf0, buf1)
    sems = (sem0, sem1)

    pltpu.async_copy(tok_hbm.at[idx_v.at[0]], bufs[0], sems[0])

    def process(g, buf, sem):
        pltpu.make_async_copy(tok_hbm.at[idx_v.at[g]], buf, sem).wait()
        pltpu.sync_copy(buf, out_hbm.at[pl.ds(wid * _ROWS_PER_W + g * _CHUNK,
                                              _CHUNK)])

    def chunk_pair(i, carry):
        g = 2 * i
        pltpu.async_copy(tok_hbm.at[idx_v.at[g + 1]], bufs[1], sems[1])
        process(g, bufs[0], sems[0])

        @pl.when(g + 2 < _NCHUNK)
        def _():
            pltpu.async_copy(tok_hbm.at[idx_v.at[g + 2]], bufs[0], sems[0])
        process(g + 1, bufs[1], sems[1])
        return carry

    lax.fori_loop(0, _NCHUNK // 2, chunk_pair, 0)


def kernel(x, token_table, pos_table):
    x3 = x.reshape(_NW, _NCHUNK, _CHUNK).astype(jnp.int32)
    mesh = plsc.VectorSubcoreMesh(core_axis_name="c", subcore_axis_name="s",
                                  num_cores=_NC, num_subcores=_NS)
    run = functools.partial(
        pl.kernel,
        out_type=jax.ShapeDtypeStruct((_BATCH * _SEQ, _DIM), jnp.float32),
        mesh=mesh,
        compiler_params=pltpu.CompilerParams(use_tc_tiling_on_sc=False),
        scratch_types=[
            pltpu.VMEM((_NCHUNK, _CHUNK), jnp.int32),
            pltpu.VMEM((_CHUNK, _DIM), jnp.float32),
            pltpu.VMEM((_CHUNK, _DIM), jnp.float32),
            pltpu.SemaphoreType.DMA,
            pltpu.SemaphoreType.DMA,
        ],
    )(_gather_kernel)
    tok_emb = run(x3, token_table)
    return tok_emb.reshape(_BATCH, _SEQ, _DIM) + pos_table[None, :_SEQ, :]
